# Initial kernel scaffold; baseline (speedup 1.0000x reference)
import jax
import jax.numpy as jnp
from jax.experimental import pallas as pl


def kernel(x_n_cat, tables):
    raise NotImplementedError("write your pallas kernel here")


# speedup vs baseline: 2.7579x; 2.7579x over previous
import functools

import jax
import jax.numpy as jnp
from jax import lax
from jax.experimental import pallas as pl
from jax.experimental.pallas import tpu as pltpu
from jax.experimental.pallas import tpu_sc as plsc

NUM_FIELDS = 26
VOCAB = 100000
HIDDEN = 128
BATCH = 16384

NC = 2
NS = 16
NW = NC * NS

TOTAL_ROWS = BATCH * NUM_FIELDS
ROWS_PER_W = TOTAL_ROWS // NW
CHUNK = 128
NCHUNKS = ROWS_PER_W // CHUNK


def _gather_body(table_hbm, idx_hbm, out_hbm, idx_v, rows_v, sem, wsem):
    wid = lax.axis_index("s") * NC + lax.axis_index("c")
    pltpu.sync_copy(idx_hbm.at[wid], idx_v)
    base = wid * ROWS_PER_W

    def chunk_step(j, _):
        pltpu.async_copy(table_hbm.at[idx_v.at[j]], rows_v, sem).wait()
        pltpu.async_copy(
            rows_v, out_hbm.at[pl.ds(base + j * CHUNK, CHUNK)], wsem
        ).wait()
        return ()

    lax.fori_loop(0, NCHUNKS, chunk_step, (), unroll=False)


@jax.jit
def _multi_embed(flat_idx, flat_table):
    mesh = plsc.VectorSubcoreMesh(
        core_axis_name="c", subcore_axis_name="s", num_cores=NC,
        num_subcores=NS,
    )
    run = pl.kernel(
        _gather_body,
        out_type=jax.ShapeDtypeStruct((TOTAL_ROWS, HIDDEN), jnp.float32),
        mesh=mesh,
        scratch_types=[
            pltpu.VMEM((NCHUNKS, CHUNK), jnp.int32),
            pltpu.VMEM((CHUNK, HIDDEN), jnp.float32),
            pltpu.SemaphoreType.DMA,
            pltpu.SemaphoreType.DMA,
        ],
    )
    return run(flat_table, flat_idx)


def kernel(x_n_cat, tables):
    flat_idx = (
        x_n_cat.astype(jnp.int32)
        + jnp.arange(NUM_FIELDS, dtype=jnp.int32)[None, :] * VOCAB
    ).reshape(NW, NCHUNKS, CHUNK)
    flat_table = tables.reshape(NUM_FIELDS * VOCAB, HIDDEN)
    out = _multi_embed(flat_idx, flat_table)
    return out.reshape(BATCH, NUM_FIELDS * HIDDEN)


# speedup vs baseline: 3.2777x; 1.1885x over previous
import functools

import jax
import jax.numpy as jnp
from jax import lax
from jax.experimental import pallas as pl
from jax.experimental.pallas import tpu as pltpu
from jax.experimental.pallas import tpu_sc as plsc

NUM_FIELDS = 26
VOCAB = 100000
HIDDEN = 128
BATCH = 16384

NC = 2
NS = 16
NW = NC * NS

TOTAL_ROWS = BATCH * NUM_FIELDS
ROWS_PER_W = TOTAL_ROWS // NW
CHUNK = 128
NCHUNKS = ROWS_PER_W // CHUNK
NBUF = 4
NGROUPS = NCHUNKS // NBUF


def _gather_body(table_hbm, idx_hbm, out_hbm, idx_v, *scratch):
    rows = scratch[:NBUF]
    gsem = scratch[NBUF:2 * NBUF]
    wsem = scratch[2 * NBUF:3 * NBUF]
    wid = lax.axis_index("s") * NC + lax.axis_index("c")
    pltpu.sync_copy(idx_hbm.at[wid], idx_v)
    base = wid * ROWS_PER_W

    def start_gather(j, b):
        pltpu.async_copy(table_hbm.at[idx_v.at[j]], rows[b], gsem[b])

    def wait_gather(j, b):
        pltpu.make_async_copy(table_hbm.at[idx_v.at[j]], rows[b],
                              gsem[b]).wait()

    def out_slice(j):
        return out_hbm.at[pl.ds(base + j * CHUNK, CHUNK)]

    def start_write(j, b):
        pltpu.async_copy(rows[b], out_slice(j), wsem[b])

    def wait_write(j, b):
        pltpu.make_async_copy(rows[b], out_slice(j), wsem[b]).wait()

    for b in range(NBUF):
        start_gather(b, b)

    def group_step(g, _):
        j0 = g * NBUF
        for b in range(NBUF):
            wait_gather(j0 + b, b)
            start_write(j0 + b, b)
        for b in range(NBUF):
            wait_write(j0 + b, b)
            start_gather(j0 + NBUF + b, b)
        return ()

    lax.fori_loop(0, NGROUPS - 1, group_step, (), unroll=False)

    j0 = (NGROUPS - 1) * NBUF
    for b in range(NBUF):
        wait_gather(j0 + b, b)
        start_write(j0 + b, b)
    for b in range(NBUF):
        wait_write(j0 + b, b)


@jax.jit
def _multi_embed(flat_idx, flat_table):
    mesh = plsc.VectorSubcoreMesh(
        core_axis_name="c", subcore_axis_name="s", num_cores=NC,
        num_subcores=NS,
    )
    run = pl.kernel(
        _gather_body,
        out_type=jax.ShapeDtypeStruct((TOTAL_ROWS, HIDDEN), jnp.float32),
        mesh=mesh,
        scratch_types=(
            [pltpu.VMEM((NCHUNKS, CHUNK), jnp.int32)]
            + [pltpu.VMEM((CHUNK, HIDDEN), jnp.float32)] * NBUF
            + [pltpu.SemaphoreType.DMA] * (2 * NBUF)
        ),
    )
    return run(flat_table, flat_idx)


def kernel(x_n_cat, tables):
    flat_idx = (
        x_n_cat.astype(jnp.int32)
        + jnp.arange(NUM_FIELDS, dtype=jnp.int32)[None, :] * VOCAB
    ).reshape(NW, NCHUNKS, CHUNK)
    flat_table = tables.reshape(NUM_FIELDS * VOCAB, HIDDEN)
    out = _multi_embed(flat_idx, flat_table)
    return out.reshape(BATCH, NUM_FIELDS * HIDDEN)


# speedup vs baseline: 3.2882x; 1.0032x over previous
import functools

import jax
import jax.numpy as jnp
from jax import lax
from jax.experimental import pallas as pl
from jax.experimental.pallas import tpu as pltpu
from jax.experimental.pallas import tpu_sc as plsc

NUM_FIELDS = 26
VOCAB = 100000
HIDDEN = 128
BATCH = 16384

NC = 2
NS = 16
NW = NC * NS

TOTAL_ROWS = BATCH * NUM_FIELDS
ROWS_PER_W = TOTAL_ROWS // NW
CHUNK = 104
NCHUNKS = ROWS_PER_W // CHUNK
NBUF = 8
NGROUPS = NCHUNKS // NBUF


def _gather_body(table_hbm, idx_hbm, out_hbm, idx_v, *scratch):
    rows = scratch[:NBUF]
    gsem = scratch[NBUF:2 * NBUF]
    wsem = scratch[2 * NBUF:3 * NBUF]
    wid = lax.axis_index("s") * NC + lax.axis_index("c")
    pltpu.sync_copy(idx_hbm.at[wid], idx_v)
    base = wid * ROWS_PER_W

    def start_gather(j, b):
        pltpu.async_copy(table_hbm.at[idx_v.at[j]], rows[b], gsem[b])

    def wait_gather(j, b):
        pltpu.make_async_copy(table_hbm.at[idx_v.at[j]], rows[b],
                              gsem[b]).wait()

    def out_slice(j):
        return out_hbm.at[pl.ds(base + j * CHUNK, CHUNK)]

    def start_write(j, b):
        pltpu.async_copy(rows[b], out_slice(j), wsem[b])

    def wait_write(j, b):
        pltpu.make_async_copy(rows[b], out_slice(j), wsem[b]).wait()

    for b in range(NBUF):
        start_gather(b, b)

    def group_step(g, _):
        j0 = g * NBUF
        for b in range(NBUF):
            wait_gather(j0 + b, b)
            start_write(j0 + b, b)
        for b in range(NBUF):
            wait_write(j0 + b, b)
            start_gather(j0 + NBUF + b, b)
        return ()

    lax.fori_loop(0, NGROUPS - 1, group_step, (), unroll=False)

    j0 = (NGROUPS - 1) * NBUF
    for b in range(NBUF):
        wait_gather(j0 + b, b)
        start_write(j0 + b, b)
    for b in range(NBUF):
        wait_write(j0 + b, b)


@jax.jit
def _multi_embed(flat_idx, flat_table):
    mesh = plsc.VectorSubcoreMesh(
        core_axis_name="c", subcore_axis_name="s", num_cores=NC,
        num_subcores=NS,
    )
    run = pl.kernel(
        _gather_body,
        out_type=jax.ShapeDtypeStruct((TOTAL_ROWS, HIDDEN), jnp.float32),
        mesh=mesh,
        scratch_types=(
            [pltpu.VMEM((NCHUNKS, CHUNK), jnp.int32)]
            + [pltpu.VMEM((CHUNK, HIDDEN), jnp.float32)] * NBUF
            + [pltpu.SemaphoreType.DMA] * (2 * NBUF)
        ),
    )
    return run(flat_table, flat_idx)


def kernel(x_n_cat, tables):
    flat_idx = (
        x_n_cat.astype(jnp.int32)
        + jnp.arange(NUM_FIELDS, dtype=jnp.int32)[None, :] * VOCAB
    ).reshape(NW, NCHUNKS, CHUNK)
    flat_table = tables.reshape(NUM_FIELDS * VOCAB, HIDDEN)
    out = _multi_embed(flat_idx, flat_table)
    return out.reshape(BATCH, NUM_FIELDS * HIDDEN)


# speedup vs baseline: 6.8027x; 2.0688x over previous
import functools

import jax
import jax.numpy as jnp
from jax import lax
from jax.experimental import pallas as pl
from jax.experimental.pallas import tpu as pltpu
from jax.experimental.pallas import tpu_sc as plsc

NUM_FIELDS = 26
VOCAB = 100000
HIDDEN = 128
BATCH = 16384

NC = 2
NS = 16
NW = NC * NS

TOTAL_ROWS = BATCH * NUM_FIELDS
ROWS_PER_W = TOTAL_ROWS // NW
CHUNK = 104
HALF_TILES = CHUNK // 8
NCHUNKS = ROWS_PER_W // CHUNK
NBUF = 8
NGROUPS = NCHUNKS // NBUF
STRIPES = BATCH // 8
STRIPES_PER_W = STRIPES // NW


def _gather_body(table_hbm, idx_hbm, out_hbm, idx_v, *scratch):
    rows = scratch[:NBUF]
    gsem = scratch[NBUF:2 * NBUF]
    wsem = scratch[2 * NBUF:3 * NBUF]
    wid = lax.axis_index("s") * NC + lax.axis_index("c")
    pltpu.sync_copy(idx_hbm.at[wid], idx_v)
    sbase = wid * STRIPES_PER_W

    def start_gather(j, b):
        pltpu.async_copy(
            table_hbm.at[idx_v.at[j]], rows[b].reshape(CHUNK, HIDDEN),
            gsem[b],
        )

    def wait_gather(j, b):
        pltpu.make_async_copy(
            table_hbm.at[idx_v.at[j]], rows[b].reshape(CHUNK, HIDDEN),
            gsem[b],
        ).wait()

    def out_slice(j):
        return out_hbm.at[
            sbase + j // 2, pl.ds((j % 2) * HALF_TILES, HALF_TILES)
        ]

    def start_write(j, b):
        pltpu.async_copy(rows[b], out_slice(j), wsem[b])

    def wait_write(j, b):
        pltpu.make_async_copy(rows[b], out_slice(j), wsem[b]).wait()

    for b in range(NBUF):
        start_gather(b, b)

    def group_step(g, _):
        j0 = g * NBUF
        for b in range(NBUF):
            wait_gather(j0 + b, b)
            start_write(j0 + b, b)
        for b in range(NBUF):
            wait_write(j0 + b, b)
            start_gather(j0 + NBUF + b, b)
        return ()

    lax.fori_loop(0, NGROUPS - 1, group_step, (), unroll=False)

    j0 = (NGROUPS - 1) * NBUF
    for b in range(NBUF):
        wait_gather(j0 + b, b)
        start_write(j0 + b, b)
    for b in range(NBUF):
        wait_write(j0 + b, b)


@jax.jit
def _multi_embed(flat_idx, flat_table):
    mesh = plsc.VectorSubcoreMesh(
        core_axis_name="c", subcore_axis_name="s", num_cores=NC,
        num_subcores=NS,
    )
    run = pl.kernel(
        _gather_body,
        out_type=jax.ShapeDtypeStruct(
            (STRIPES, NUM_FIELDS, 8, HIDDEN), jnp.float32
        ),
        mesh=mesh,
        scratch_types=(
            [pltpu.VMEM((NCHUNKS, CHUNK), jnp.int32)]
            + [pltpu.VMEM((HALF_TILES, 8, HIDDEN), jnp.float32)] * NBUF
            + [pltpu.SemaphoreType.DMA] * (2 * NBUF)
        ),
    )
    return run(flat_table, flat_idx)


def kernel(x_n_cat, tables):
    flat = (
        x_n_cat.astype(jnp.int32)
        + jnp.arange(NUM_FIELDS, dtype=jnp.int32)[None, :] * VOCAB
    )
    perm = flat.reshape(NW, STRIPES_PER_W, 8, NUM_FIELDS).transpose(
        0, 1, 3, 2
    )
    flat_idx = perm.reshape(NW, NCHUNKS, CHUNK)
    flat_table = tables.reshape(NUM_FIELDS * VOCAB, HIDDEN)
    out4 = _multi_embed(flat_idx, flat_table)
    return out4.transpose(0, 2, 1, 3).reshape(BATCH, NUM_FIELDS * HIDDEN)


# speedup vs baseline: 6.8177x; 1.0022x over previous
import functools

import jax
import jax.numpy as jnp
from jax import lax
from jax.experimental import pallas as pl
from jax.experimental.pallas import tpu as pltpu
from jax.experimental.pallas import tpu_sc as plsc

NUM_FIELDS = 26
VOCAB = 100000
HIDDEN = 128
BATCH = 16384

NC = 2
NS = 16
NW = NC * NS

TOTAL_ROWS = BATCH * NUM_FIELDS
ROWS_PER_W = TOTAL_ROWS // NW
CHUNK = 104
HALF_TILES = CHUNK // 8
NCHUNKS = ROWS_PER_W // CHUNK
NBUF = 8
NGROUPS = NCHUNKS // NBUF
STRIPES = BATCH // 8
STRIPES_PER_W = STRIPES // NW
SROWS = 8 * NUM_FIELDS


def _gather_body(table_hbm, idx_hbm, out_hbm, idx_v, *scratch):
    rows = scratch[:NBUF]
    gsem = scratch[NBUF:2 * NBUF]
    wsem = scratch[2 * NBUF:3 * NBUF]
    wid = lax.axis_index("s") * NC + lax.axis_index("c")
    pltpu.sync_copy(idx_hbm.at[wid], idx_v)
    sbase = wid * STRIPES_PER_W

    def start_gather(j, b):
        pltpu.async_copy(
            table_hbm.at[idx_v.at[j]], rows[b].reshape(CHUNK, HIDDEN),
            gsem[b],
        )

    def wait_gather(j, b):
        pltpu.make_async_copy(
            table_hbm.at[idx_v.at[j]], rows[b].reshape(CHUNK, HIDDEN),
            gsem[b],
        ).wait()

    def out_slice(j):
        return out_hbm.at[
            sbase + j // 2, pl.ds((j % 2) * HALF_TILES, HALF_TILES)
        ]

    def start_write(j, b):
        pltpu.async_copy(rows[b], out_slice(j), wsem[b])

    def wait_write(j, b):
        pltpu.make_async_copy(rows[b], out_slice(j), wsem[b]).wait()

    for b in range(NBUF):
        start_gather(b, b)

    def group_step(g, _):
        j0 = g * NBUF
        for b in range(NBUF):
            wait_gather(j0 + b, b)
            start_write(j0 + b, b)
        for b in range(NBUF):
            wait_write(j0 + b, b)
            start_gather(j0 + NBUF + b, b)
        return ()

    lax.fori_loop(0, NGROUPS - 1, group_step, (), unroll=False)

    j0 = (NGROUPS - 1) * NBUF
    for b in range(NBUF):
        wait_gather(j0 + b, b)
        start_write(j0 + b, b)
    for b in range(NBUF):
        wait_write(j0 + b, b)


@jax.jit
def _multi_embed(flat_idx, flat_table):
    mesh = plsc.VectorSubcoreMesh(
        core_axis_name="c", subcore_axis_name="s", num_cores=NC,
        num_subcores=NS,
    )
    run = pl.kernel(
        _gather_body,
        out_type=jax.ShapeDtypeStruct(
            (STRIPES, NUM_FIELDS, 8, HIDDEN), jnp.float32
        ),
        mesh=mesh,
        scratch_types=(
            [pltpu.VMEM((NCHUNKS, CHUNK), jnp.int32)]
            + [pltpu.VMEM((HALF_TILES, 8, HIDDEN), jnp.float32)] * NBUF
            + [pltpu.SemaphoreType.DMA] * (2 * NBUF)
        ),
    )
    return run(flat_table, flat_idx)


def kernel(x_n_cat, tables):
    flat = (
        x_n_cat.astype(jnp.int32)
        + jnp.arange(NUM_FIELDS, dtype=jnp.int32)[None, :] * VOCAB
    )
    perm = flat.reshape(NW, STRIPES_PER_W, 8, NUM_FIELDS).transpose(
        0, 1, 3, 2
    )
    flat_idx = perm.reshape(NW, NCHUNKS, CHUNK)
    flat_table = tables.reshape(NUM_FIELDS * VOCAB, HIDDEN)
    out4 = _multi_embed(flat_idx, flat_table)
    return out4.transpose(0, 2, 1, 3).reshape(BATCH, NUM_FIELDS * HIDDEN)
